# baseline (speedup 1.0000x reference)
import functools

import jax
import jax.numpy as jnp
from jax import lax
from jax.experimental import pallas as pl
from jax.experimental.pallas import tpu as pltpu
from jax.experimental.pallas import tpu_sc as plsc

B = 256
C, H, W = 3, 224, 224
D = C * H * W
NCLS = 1000
ALPHA = 0.2

NC, NS, L = 2, 16, 16
NW = NC * NS

NCH = 49
CH = D // NCH
ROWS = B * NCH
RPW = ROWS // NW
K = 4
NB = RPW // K

YC = 1024
YPW = B // NW

_mesh = plsc.VectorSubcoreMesh(core_axis_name="c", subcore_axis_name="s")


@functools.partial(
    pl.kernel,
    mesh=_mesh,
    out_type=(
        jax.ShapeDtypeStruct((ROWS, CH), jnp.float32),
        jax.ShapeDtypeStruct((B, YC), jnp.float32),
    ),
    scratch_types=[
        pltpu.VMEM((NB, K), jnp.int32),
        pltpu.VMEM((NB, K), jnp.int32),
        pltpu.VMEM((K, CH), jnp.float32),
        pltpu.VMEM((K, CH), jnp.float32),
        pltpu.VMEM((K, CH), jnp.float32),
        pltpu.VMEM((RPW * L,), jnp.float32),
        pltpu.VMEM((YPW,), jnp.int32),
        pltpu.VMEM((YPW,), jnp.int32),
        pltpu.VMEM((YPW, YC), jnp.float32),
        pltpu.VMEM((YPW, YC), jnp.float32),
        pltpu.VMEM((YPW, YC), jnp.float32),
        pltpu.VMEM((YPW * L,), jnp.float32),
        pltpu.SemaphoreType.DMA,
        pltpu.SemaphoreType.DMA,
    ],
)
def _mix_kernel(xr, src1, src2, lamc, yp, ysrc1, ysrc2, ylam,
                out_x, out_y,
                i1_v, i2_v, a_v, b_v, o_v, lam_v,
                yi1_v, yi2_v, ya_v, yb_v, yo_v, ylam_v,
                sem_a, sem_b):
    wid = lax.axis_index("s") * NC + lax.axis_index("c")
    base = wid * RPW

    pltpu.sync_copy(src1.at[wid], i1_v)
    pltpu.sync_copy(src2.at[wid], i2_v)
    pltpu.sync_copy(lamc.at[wid], lam_v)

    def x_iter(t, carry):
        off = t * K
        cp_a = pltpu.make_async_copy(xr.at[i1_v.at[t]], a_v, sem_a)
        cp_b = pltpu.make_async_copy(xr.at[i2_v.at[t]], b_v, sem_b)
        cp_a.start()
        cp_b.start()
        cp_a.wait()
        cp_b.wait()
        for r in range(K):
            lam = lam_v[pl.ds(pl.multiple_of((off + r) * L, L), L)]

            def col(j, _, r=r, lam=lam):
                a = a_v[r, pl.ds(j * L, L)]
                b = b_v[r, pl.ds(j * L, L)]
                o_v[r, pl.ds(j * L, L)] = b + lam * (a - b)
                return _

            lax.fori_loop(0, CH // L, col, 0, unroll=4)
        pltpu.sync_copy(o_v, out_x.at[pl.ds(base + off, K)])
        return carry

    lax.fori_loop(0, NB, x_iter, 0)

    ybase = wid * YPW
    pltpu.sync_copy(ysrc1.at[pl.ds(ybase, YPW)], yi1_v)
    pltpu.sync_copy(ysrc2.at[pl.ds(ybase, YPW)], yi2_v)
    pltpu.sync_copy(ylam.at[wid], ylam_v)
    ycp_a = pltpu.make_async_copy(yp.at[yi1_v], ya_v, sem_a)
    ycp_b = pltpu.make_async_copy(yp.at[yi2_v], yb_v, sem_b)
    ycp_a.start()
    ycp_b.start()
    ycp_a.wait()
    ycp_b.wait()
    for r in range(YPW):
        lam = ylam_v[pl.ds(r * L, L)]

        def ycol(j, _, r=r, lam=lam):
            a = ya_v[r, pl.ds(j * L, L)]
            b = yb_v[r, pl.ds(j * L, L)]
            yo_v[r, pl.ds(j * L, L)] = b + lam * (a - b)
            return _

        lax.fori_loop(0, YC // L, ycol, 0, unroll=4)
    pltpu.sync_copy(yo_v, out_y.at[pl.ds(ybase, YPW)])


def kernel(x, y):
    key = jax.random.key(42)
    k1, k2, k3 = jax.random.split(key, 3)
    idx_1 = jax.random.permutation(k1, B)
    idx_2 = jax.random.permutation(k2, B)
    lam = jax.random.beta(k3, ALPHA, ALPHA, (B,)).astype(jnp.float32)

    carange = jnp.arange(NCH, dtype=jnp.int32)
    src1 = (idx_1.astype(jnp.int32)[:, None] * NCH + carange).reshape(NW, NB, K)
    src2 = (idx_2.astype(jnp.int32)[:, None] * NCH + carange).reshape(NW, NB, K)
    lamc = jnp.broadcast_to(lam[:, None, None], (B, NCH, L)).reshape(NW, RPW * L)
    ylamr = jnp.broadcast_to(lam[:, None], (B, L)).reshape(NW, YPW * L)

    xr = x.reshape(ROWS, CH)
    yp = jnp.pad(y.reshape(B, NCLS), ((0, 0), (0, YC - NCLS)))

    out_x, out_y = _mix_kernel(
        xr, src1, src2, lamc,
        yp, idx_1.astype(jnp.int32), idx_2.astype(jnp.int32), ylamr,
    )
    return (out_x.reshape(B, C, H, W), out_y[:, :NCLS].reshape(B, NCLS, 1, 1))
